# Initial kernel scaffold; baseline (speedup 1.0000x reference)
import jax
import jax.numpy as jnp
from jax.experimental import pallas as pl


def kernel(token, W0, b0, Wa1, ba1, Wb1, bb1, W1, b1, Wa2, ba2, Wb2, bb2, Wout, bout, art_codebooks, pitch_codebooks):
    raise NotImplementedError("write your pallas kernel here")


# speedup vs baseline: 1.0385x; 1.0385x over previous
import functools

import jax
import jax.numpy as jnp
from jax.experimental import pallas as pl
from jax.experimental.pallas import tpu as pltpu

B, T, D_IN, D_HID, D_OUT = 8, 1024, 512, 512, 256
PITCH_DIM = 8
ART_Q, ART_K, ART_D = 4, 1024, 248
PIT_Q, PIT_K, PIT_D = 2, 256, 8
N_TOK = B * T
BT = 512
GRID = (N_TOK // BT,)


def _dot(a, b):
    return jnp.dot(a, b, preferred_element_type=jnp.float32)


def _rowdot(a, b):
    return jax.lax.dot_general(
        a, b, dimension_numbers=(((1,), (1,)), ((), ())),
        preferred_element_type=jnp.float32)


def _split3(cb):
    mask = jnp.uint32(0xFFFF0000)
    trunc = lambda v: jax.lax.bitcast_convert_type(
        jax.lax.bitcast_convert_type(v, jnp.uint32) & mask, jnp.float32)
    hi = trunc(cb)
    r1 = cb - hi
    mid = trunc(r1)
    lo = r1 - mid
    return hi, mid, lo


def _encoder_kernel(x_ref, W0_ref, b0_ref, Wa1_ref, ba1_ref, Wb1_ref, bb1_ref,
                    W1_ref, b1_ref, Wa2_ref, ba2_ref, Wb2_ref, bb2_ref,
                    Wout_ref, bout_ref, t_ref):
    h = _dot(x_ref[...], W0_ref[...]) + b0_ref[...]
    h = jnp.maximum(_dot(h, Wa1_ref[...]) + ba1_ref[...], 0.0)
    h = _dot(h, Wb1_ref[...]) + bb1_ref[...]
    h = _dot(h, W1_ref[...]) + b1_ref[...]
    h = jnp.maximum(_dot(h, Wa2_ref[...]) + ba2_ref[...], 0.0)
    h = _dot(h, Wb2_ref[...]) + bb2_ref[...]
    t_ref[...] = _dot(h, Wout_ref[...]) + bout_ref[...]


def _vq_stage_kernel(kdim, r_ref, r2_ref, cb_ref, hi_ref, mid_ref, lo_ref,
                     c2_ref, ind_ref, q_ref, rn_ref):
    r = r_ref[...]
    m = _rowdot(r, cb_ref[...])
    dist = (r2_ref[...] - 2.0 * m) + c2_ref[...]
    md = jnp.min(dist, axis=1, keepdims=True)
    kiota = jax.lax.broadcasted_iota(jnp.int32, dist.shape, 1)
    ind = jnp.min(jnp.where(dist == md, kiota, kdim), axis=1, keepdims=True)
    onehot = (kiota == ind).astype(jnp.float32)
    q = (_dot(onehot, hi_ref[...]) + _dot(onehot, mid_ref[...])) \
        + _dot(onehot, lo_ref[...])
    ind_ref[...] = ind
    q_ref[...] = q
    rn_ref[...] = r - q


def _final_kernel(aq_ref, p1_ref, p2_ref, quant_ref):
    pacc = p1_ref[...] + p2_ref[...]
    q = jnp.concatenate([aq_ref[...], pacc], axis=1)
    qa = q[:, :ART_D]
    qp = q[:, ART_D:]
    na = jnp.sqrt((qa ** 2).sum(-1, keepdims=True) + 1e-5)
    na = jnp.where(na == 0.0, 1.0, na)
    npn = jnp.sqrt((qp ** 2).sum(-1, keepdims=True) + 1e-5)
    npn = jnp.where(npn == 0.0, 1.0, npn)
    quant_ref[...] = jnp.concatenate([qa / na, qp / npn], axis=1)


_CP = pltpu.CompilerParams(dimension_semantics=("arbitrary",))


def _vq_stage(r, r2, cb, hi, mid, lo, c2, kdim, ddim, interpret):
    row_spec = lambda w: pl.BlockSpec((BT, w), lambda i: (i, 0))
    w_spec = lambda a: pl.BlockSpec(a.shape, lambda i: (0,) * a.ndim)
    return pl.pallas_call(
        functools.partial(_vq_stage_kernel, kdim),
        grid=GRID,
        in_specs=[row_spec(ddim), row_spec(1), w_spec(cb), w_spec(hi),
                  w_spec(mid), w_spec(lo), w_spec(c2)],
        out_specs=[row_spec(1), row_spec(ddim), row_spec(ddim)],
        out_shape=[
            jax.ShapeDtypeStruct((N_TOK, 1), jnp.int32),
            jax.ShapeDtypeStruct((N_TOK, ddim), jnp.float32),
            jax.ShapeDtypeStruct((N_TOK, ddim), jnp.float32),
        ],
        compiler_params=_CP,
        interpret=interpret,
    )(r, r2, cb, hi, mid, lo, c2)


def _unit_norm(x):
    norm = jnp.sqrt((x ** 2).sum(-1, keepdims=True) + 1e-05)
    norm = jnp.where(norm == 0, 1.0, norm)
    return x / norm


@functools.partial(jax.jit, static_argnames=("interpret",))
def kernel(token, W0, b0, Wa1, ba1, Wb1, bb1, W1, b1, Wa2, ba2, Wb2, bb2,
           Wout, bout, art_codebooks, pitch_codebooks, interpret=False):
    non_blank_mask = (token ** 2).sum(-1) > 0
    x = _unit_norm(token).reshape(N_TOK, D_IN)

    row2 = lambda v: v.reshape(1, -1)
    row_spec = lambda w: pl.BlockSpec((BT, w), lambda i: (i, 0))
    w_spec = lambda a: pl.BlockSpec(a.shape, lambda i: (0,) * a.ndim)
    enc_args = (W0, row2(b0), Wa1, row2(ba1), Wb1, row2(bb1), W1, row2(b1),
                Wa2, row2(ba2), Wb2, row2(bb2), Wout, row2(bout))
    t_pre = pl.pallas_call(
        _encoder_kernel,
        grid=GRID,
        in_specs=[row_spec(D_IN)] + [w_spec(a) for a in enc_args],
        out_specs=row_spec(D_OUT),
        out_shape=jax.ShapeDtypeStruct((N_TOK, D_OUT), jnp.float32),
        compiler_params=_CP,
        interpret=interpret,
    )(x, *enc_args)

    t = jnp.concatenate(
        [_unit_norm(t_pre[..., :-PITCH_DIM]), _unit_norm(t_pre[..., -PITCH_DIM:])], -1)
    t = jnp.where(non_blank_mask.reshape(N_TOK)[..., None], t, 0.0)

    inds = []
    loss = jnp.asarray(0.0, jnp.float32)

    def run_vq(res, cbs, kdim, ddim):
        nonlocal loss, inds
        qacc = None
        for i in range(cbs.shape[0]):
            cb = cbs[i]
            hi, mid, lo = _split3(cb)
            c2 = (cb ** 2).sum(-1).reshape(1, kdim)
            r2 = (res ** 2).sum(-1, keepdims=True)
            ind, q, res = _vq_stage(res, r2, cb, hi, mid, lo, c2, kdim, ddim,
                                    interpret)
            inds.append(ind)
            qacc = q if qacc is None else qacc + q
            loss = loss + jnp.mean(res ** 2)
        return qacc, res

    art_q, _ = run_vq(t[:, :ART_D], art_codebooks, ART_K, ART_D)
    pit_res = t[:, ART_D:]
    p_qs = []
    for j in range(PIT_Q):
        cb = pitch_codebooks[j]
        hi, mid, lo = _split3(cb)
        c2 = (cb ** 2).sum(-1).reshape(1, PIT_K)
        r2 = (pit_res ** 2).sum(-1, keepdims=True)
        ind, q, pit_res = _vq_stage(pit_res, r2, cb, hi, mid, lo, c2, PIT_K,
                                    PIT_D, interpret)
        inds.append(ind)
        p_qs.append(q)
        loss = loss + jnp.mean(pit_res ** 2)

    quantized = pl.pallas_call(
        _final_kernel,
        grid=GRID,
        in_specs=[row_spec(ART_D), row_spec(PIT_D), row_spec(PIT_D)],
        out_specs=row_spec(D_OUT),
        out_shape=jax.ShapeDtypeStruct((N_TOK, D_OUT), jnp.float32),
        compiler_params=_CP,
        interpret=interpret,
    )(art_q, p_qs[0], p_qs[1])

    indices = jnp.concatenate(inds, axis=1).reshape(B, T, ART_Q + PIT_Q)
    return (indices, quantized.reshape(B, T, D_OUT),
            t.reshape(B, T, D_OUT), loss)
